# baseline (speedup 1.0000x reference)
import functools

import jax
import jax.numpy as jnp
from jax import lax
from jax.experimental import pallas as pl
from jax.experimental.pallas import tpu as pltpu
from jax.experimental.pallas import tpu_sc as plsc

VOCAB = 100000
NPOS = 8192
DMODEL = 1024
BATCH = 4
SEQ = 2048

B = BATCH * SEQ
NW = 32
ROWS_PER_W = B // NW
CHUNK = 16
NCHUNK = ROWS_PER_W // CHUNK
LANES = 16
VECS_PER_CHUNK = CHUNK * DMODEL // LANES

_mesh = plsc.VectorSubcoreMesh(core_axis_name="c", subcore_axis_name="s")


@functools.partial(
    pl.kernel,
    mesh=_mesh,
    out_type=jax.ShapeDtypeStruct((B, DMODEL), jnp.float32),
    scratch_types=[
        pltpu.VMEM((CHUNK,), jnp.int32),
        pltpu.VMEM((CHUNK,), jnp.int32),
        pltpu.VMEM((CHUNK, DMODEL), jnp.float32),
        pltpu.VMEM((CHUNK, DMODEL), jnp.float32),
        pltpu.SemaphoreType.DMA,
        pltpu.SemaphoreType.DMA,
    ],
)
def _emb_kernel(tok_hbm, pos_hbm, wte_hbm, wpe_hbm, out_hbm,
                tok_v, pos_v, a_v, b_v, sem_a, sem_b):
    wid = lax.axis_index("s") * 2 + lax.axis_index("c")
    base = wid * ROWS_PER_W

    def chunk_body(ci, _):
        rbase = base + ci * CHUNK
        pltpu.sync_copy(tok_hbm.at[pl.ds(rbase, CHUNK)], tok_v)
        pltpu.sync_copy(pos_hbm.at[pl.ds(rbase, CHUNK)], pos_v)
        cp_a = pltpu.async_copy(wte_hbm.at[tok_v], a_v, sem_a)
        cp_b = pltpu.async_copy(wpe_hbm.at[pos_v], b_v, sem_b)
        cp_a.wait()
        cp_b.wait()

        def add_body(i, _):
            r = i // (DMODEL // LANES)
            c = (i % (DMODEL // LANES)) * LANES
            a_v[r, pl.ds(c, LANES)] = a_v[r, pl.ds(c, LANES)] + b_v[r, pl.ds(c, LANES)]
            return 0

        lax.fori_loop(0, VECS_PER_CHUNK, add_body, 0)
        pltpu.sync_copy(a_v, out_hbm.at[pl.ds(rbase, CHUNK)])
        return 0

    lax.fori_loop(0, NCHUNK, chunk_body, 0)


def kernel(input_ids, position_ids, wte, wpe):
    tok = input_ids.reshape(B).astype(jnp.int32)
    pos = position_ids.reshape(B).astype(jnp.int32)
    out = _emb_kernel(tok, pos, wte, wpe)
    return out.reshape(BATCH, SEQ, DMODEL)


# speedup vs baseline: 1.7773x; 1.7773x over previous
import functools

import jax
import jax.numpy as jnp
from jax import lax
from jax.experimental import pallas as pl
from jax.experimental.pallas import tpu as pltpu
from jax.experimental.pallas import tpu_sc as plsc

VOCAB = 100000
NPOS = 8192
DMODEL = 1024
BATCH = 4
SEQ = 2048

B = BATCH * SEQ
NW = 32
ROWS_PER_W = B // NW
CHUNK = 16
NCHUNK = ROWS_PER_W // CHUNK
LANES = 16
COLB = DMODEL // LANES

_mesh = plsc.VectorSubcoreMesh(core_axis_name="c", subcore_axis_name="s")


@functools.partial(
    pl.kernel,
    mesh=_mesh,
    out_type=jax.ShapeDtypeStruct((B, DMODEL), jnp.float32),
    scratch_types=[
        pltpu.VMEM((ROWS_PER_W,), jnp.int32),
        pltpu.VMEM((ROWS_PER_W,), jnp.int32),
        pltpu.VMEM((CHUNK, DMODEL), jnp.float32),
        pltpu.VMEM((CHUNK, DMODEL), jnp.float32),
        pltpu.VMEM((CHUNK, DMODEL), jnp.float32),
        pltpu.VMEM((CHUNK, DMODEL), jnp.float32),
        pltpu.SemaphoreType.DMA,
        pltpu.SemaphoreType.DMA,
        pltpu.SemaphoreType.DMA,
        pltpu.SemaphoreType.DMA,
        pltpu.SemaphoreType.DMA,
        pltpu.SemaphoreType.DMA,
    ],
)
def _emb_kernel(tok_hbm, pos_hbm, wte_hbm, wpe_hbm, out_hbm,
                tok_v, pos_v, a0, a1, b0, b1,
                sga0, sga1, sgb0, sgb1, sst0, sst1):
    wid = lax.axis_index("s") * 2 + lax.axis_index("c")
    base = wid * ROWS_PER_W

    a_bufs, b_bufs = (a0, a1), (b0, b1)
    sga, sgb, sst = (sga0, sga1), (sgb0, sgb1), (sst0, sst1)

    pltpu.sync_copy(tok_hbm.at[pl.ds(base, ROWS_PER_W)], tok_v)
    pltpu.sync_copy(pos_hbm.at[pl.ds(base, ROWS_PER_W)], pos_v)

    def issue_gathers(ci):
        p = ci % 2
        off = ci * CHUNK
        cpa = pltpu.async_copy(
            wte_hbm.at[tok_v.at[pl.ds(off, CHUNK)]], a_bufs[p], sga[p])
        cpb = pltpu.async_copy(
            wpe_hbm.at[pos_v.at[pl.ds(off, CHUNK)]], b_bufs[p], sgb[p])
        return cpa, cpb

    def add_into(a_buf, b_buf):
        def body(j, _):
            c = j * LANES
            for r in range(CHUNK):
                plsc.addupdate(a_buf.at[r, pl.ds(c, LANES)],
                               b_buf[r, pl.ds(c, LANES)])
            return 0
        lax.fori_loop(0, COLB, body, 0)

    gat = {0: issue_gathers(0)}
    sto = {}
    for ci in range(NCHUNK):
        p = ci % 2
        if ci + 1 < NCHUNK:
            if ci >= 1:
                sto[ci - 1].wait()
            gat[ci + 1] = issue_gathers(ci + 1)
        cpa, cpb = gat.pop(ci)
        cpa.wait()
        cpb.wait()
        add_into(a_bufs[p], b_bufs[p])
        sto[ci] = pltpu.async_copy(
            a_bufs[p], out_hbm.at[pl.ds(base + ci * CHUNK, CHUNK)], sst[p])
    sto[NCHUNK - 2].wait()
    sto[NCHUNK - 1].wait()


def kernel(input_ids, position_ids, wte, wpe):
    tok = input_ids.reshape(B).astype(jnp.int32)
    pos = position_ids.reshape(B).astype(jnp.int32)
    out = _emb_kernel(tok, pos, wte, wpe)
    return out.reshape(BATCH, SEQ, DMODEL)


# speedup vs baseline: 1.7995x; 1.0125x over previous
import functools

import jax
import jax.numpy as jnp
from jax import lax
from jax.experimental import pallas as pl
from jax.experimental.pallas import tpu as pltpu
from jax.experimental.pallas import tpu_sc as plsc

VOCAB = 100000
NPOS = 8192
DMODEL = 1024
BATCH = 4
SEQ = 2048

B = BATCH * SEQ
NW = 32
ROWS_PER_W = B // NW
CHUNK = 16
NCHUNK = ROWS_PER_W // CHUNK
NBUF = 3
LANES = 16
COLB = DMODEL // LANES

_mesh = plsc.VectorSubcoreMesh(core_axis_name="c", subcore_axis_name="s")


@functools.partial(
    pl.kernel,
    mesh=_mesh,
    out_type=jax.ShapeDtypeStruct((B, DMODEL), jnp.float32),
    scratch_types=[
        pltpu.VMEM((ROWS_PER_W,), jnp.int32),
        pltpu.VMEM((ROWS_PER_W,), jnp.int32),
        pltpu.VMEM((NBUF, CHUNK, DMODEL), jnp.float32),
        pltpu.VMEM((NBUF, CHUNK, DMODEL), jnp.float32),
        pltpu.SemaphoreType.DMA,
        pltpu.SemaphoreType.DMA,
        pltpu.SemaphoreType.DMA,
        pltpu.SemaphoreType.DMA,
        pltpu.SemaphoreType.DMA,
        pltpu.SemaphoreType.DMA,
        pltpu.SemaphoreType.DMA,
        pltpu.SemaphoreType.DMA,
        pltpu.SemaphoreType.DMA,
        pltpu.SemaphoreType.DMA,
        pltpu.SemaphoreType.DMA,
    ],
)
def _emb_kernel(tok_hbm, pos_hbm, wte_hbm, wpe_hbm, out_hbm,
                tok_v, pos_v, a_v, b_v,
                sit, sip, sga0, sga1, sga2, sgb0, sgb1, sgb2,
                sst0, sst1, sst2):
    wid = lax.axis_index("s") * 2 + lax.axis_index("c")
    base = wid * ROWS_PER_W

    sga, sgb, sst = (sga0, sga1, sga2), (sgb0, sgb1, sgb2), (sst0, sst1, sst2)

    cit = pltpu.async_copy(tok_hbm.at[pl.ds(base, ROWS_PER_W)], tok_v, sit)
    cip = pltpu.async_copy(pos_hbm.at[pl.ds(base, ROWS_PER_W)], pos_v, sip)
    cit.wait()
    cip.wait()

    def issue_gathers(ci):
        p = ci % NBUF
        off = ci * CHUNK
        cpa = pltpu.async_copy(
            wte_hbm.at[tok_v.at[pl.ds(off, CHUNK)]], a_v.at[p], sga[p])
        cpb = pltpu.async_copy(
            wpe_hbm.at[pos_v.at[pl.ds(off, CHUNK)]], b_v.at[p], sgb[p])
        return cpa, cpb

    def add_into(p):
        def body(j, _):
            c = j * LANES
            for r in range(CHUNK):
                plsc.addupdate(a_v.at[p, r, pl.ds(c, LANES)],
                               b_v[p, r, pl.ds(c, LANES)])
            return 0
        lax.fori_loop(0, COLB, body, 0)

    gat = {0: issue_gathers(0), 1: issue_gathers(1)}
    sto = {}
    for ci in range(NCHUNK):
        p = ci % NBUF
        if ci + 2 < NCHUNK:
            if ci >= 1:
                sto[ci - 1].wait()
            gat[ci + 2] = issue_gathers(ci + 2)
        cpa, cpb = gat.pop(ci)
        cpa.wait()
        cpb.wait()
        add_into(p)
        sto[ci] = pltpu.async_copy(
            a_v.at[p], out_hbm.at[pl.ds(base + ci * CHUNK, CHUNK)], sst[p])
    for ci in range(NCHUNK - NBUF, NCHUNK):
        sto[ci].wait()


def kernel(input_ids, position_ids, wte, wpe):
    tok = input_ids.reshape(B).astype(jnp.int32)
    pos = position_ids.reshape(B).astype(jnp.int32)
    out = _emb_kernel(tok, pos, wte, wpe)
    return out.reshape(BATCH, SEQ, DMODEL)


# speedup vs baseline: 2.2117x; 1.2291x over previous
import functools

import jax
import jax.numpy as jnp
from jax import lax
from jax.experimental import pallas as pl
from jax.experimental.pallas import tpu as pltpu
from jax.experimental.pallas import tpu_sc as plsc

VOCAB = 100000
NPOS = 8192
DMODEL = 1024
BATCH = 4
SEQ = 2048

B = BATCH * SEQ
NW = 32
ROWS_PER_W = B // NW
CHUNK = 16
NCHUNK = ROWS_PER_W // CHUNK
NROUND = NCHUNK // 2
LANES = 16
COLB = DMODEL // LANES

_mesh = plsc.VectorSubcoreMesh(core_axis_name="c", subcore_axis_name="s")


@functools.partial(
    pl.kernel,
    mesh=_mesh,
    out_type=jax.ShapeDtypeStruct((B, DMODEL), jnp.float32),
    scratch_types=[
        pltpu.VMEM((ROWS_PER_W,), jnp.int32),
        pltpu.VMEM((ROWS_PER_W,), jnp.int32),
        pltpu.VMEM((CHUNK, DMODEL), jnp.float32),
        pltpu.VMEM((CHUNK, DMODEL), jnp.float32),
        pltpu.VMEM((CHUNK, DMODEL), jnp.float32),
        pltpu.VMEM((CHUNK, DMODEL), jnp.float32),
        pltpu.SemaphoreType.DMA,
        pltpu.SemaphoreType.DMA,
        pltpu.SemaphoreType.DMA,
        pltpu.SemaphoreType.DMA,
        pltpu.SemaphoreType.DMA,
        pltpu.SemaphoreType.DMA,
        pltpu.SemaphoreType.DMA,
        pltpu.SemaphoreType.DMA,
    ],
)
def _emb_kernel(tok_hbm, pos_hbm, wte_hbm, wpe_hbm, out_hbm,
                tok_v, pos_v, a0, a1, b0, b1,
                sit, sip, sga0, sga1, sgb0, sgb1, sst0, sst1):
    wid = lax.axis_index("s") * 2 + lax.axis_index("c")
    base = wid * ROWS_PER_W

    a_bufs, b_bufs = (a0, a1), (b0, b1)
    sga, sgb, sst = (sga0, sga1), (sgb0, sgb1), (sst0, sst1)

    cit = pltpu.async_copy(tok_hbm.at[pl.ds(base, ROWS_PER_W)], tok_v, sit)
    cip = pltpu.async_copy(pos_hbm.at[pl.ds(base, ROWS_PER_W)], pos_v, sip)
    cit.wait()
    cip.wait()

    def issue_g(ci, p):
        off = ci * CHUNK
        pltpu.async_copy(
            wte_hbm.at[tok_v.at[pl.ds(off, CHUNK)]], a_bufs[p], sga[p])
        pltpu.async_copy(
            wpe_hbm.at[pos_v.at[pl.ds(off, CHUNK)]], b_bufs[p], sgb[p])

    def wait_g(p):
        pltpu.make_async_copy(
            wte_hbm.at[pl.ds(0, CHUNK)], a_bufs[p], sga[p]).wait()
        pltpu.make_async_copy(
            wpe_hbm.at[pl.ds(0, CHUNK)], b_bufs[p], sgb[p]).wait()

    def issue_s(ci, p):
        pltpu.async_copy(
            a_bufs[p], out_hbm.at[pl.ds(base + ci * CHUNK, CHUNK)], sst[p])

    def wait_s(p):
        pltpu.make_async_copy(
            a_bufs[p], out_hbm.at[pl.ds(base, CHUNK)], sst[p]).wait()

    def add_into(p):
        a_buf, b_buf = a_bufs[p], b_bufs[p]

        @plsc.parallel_loop(0, COLB, 1, unroll=4)
        def _(j):
            c = j * LANES
            for r in range(CHUNK):
                plsc.addupdate(a_buf.at[r, pl.ds(c, LANES)],
                               b_buf[r, pl.ds(c, LANES)])

    issue_g(0, 0)
    issue_g(1, 1)

    def round_body(i, _):
        c0 = 2 * i
        wait_g(0)
        add_into(0)
        issue_s(c0, 0)
        wait_g(1)
        add_into(1)
        issue_s(c0 + 1, 1)

        @pl.when(i < NROUND - 1)
        def _prefetch():
            wait_s(0)
            issue_g(c0 + 2, 0)
            wait_s(1)
            issue_g(c0 + 3, 1)

        return 0

    lax.fori_loop(0, NROUND, round_body, 0)
    wait_s(0)
    wait_s(1)


def kernel(input_ids, position_ids, wte, wpe):
    tok = input_ids.reshape(B).astype(jnp.int32)
    pos = position_ids.reshape(B).astype(jnp.int32)
    out = _emb_kernel(tok, pos, wte, wpe)
    return out.reshape(BATCH, SEQ, DMODEL)
